# baseline (speedup 1.0000x reference)
import functools

import jax
import jax.numpy as jnp
from jax import lax
from jax.experimental import pallas as pl
from jax.experimental.pallas import tpu as pltpu
from jax.experimental.pallas import tpu_sc as plsc

_CHUNK = 128
_LANES = 16


@functools.lru_cache(maxsize=None)
def _make_convert(V, D):
    info = plsc.get_sparse_core_info()
    NC, NS = info.num_cores, info.num_subcores
    NW = NC * NS
    n_tiles = (V + _CHUNK - 1) // _CHUNK
    per_w = (n_tiles + NW - 1) // NW
    rows_out = n_tiles * D
    mesh = plsc.VectorSubcoreMesh(core_axis_name="c", subcore_axis_name="s")

    @functools.partial(
        pl.kernel,
        mesh=mesh,
        out_type=jax.ShapeDtypeStruct((rows_out, _CHUNK), jnp.float32),
        compiler_params=pltpu.CompilerParams(needs_layout_passes=False),
        scratch_types=[
            pltpu.VMEM((D, _CHUNK), jnp.float32),
            pltpu.VMEM((D, _CHUNK), jnp.float32),
            pltpu.VMEM((D, _CHUNK), jnp.float32),
            pltpu.SemaphoreType.DMA,
            pltpu.SemaphoreType.DMA,
        ],
    )
    def convert_kernel(tblT_hbm, cvt_hbm, src0, src1, dst_v, sem0, sem1):
        wid = lax.axis_index("s") * NC + lax.axis_index("c")
        start = jnp.minimum(wid * per_w, n_tiles - per_w)
        srcs = (src0, src1)
        sems = (sem0, sem1)
        lane = lax.broadcasted_iota(jnp.int32, (_LANES,), 0)
        f_lo = lane
        f_hi = lane + _LANES

        def tile_of(t):
            return start + jnp.minimum(t, per_w - 1)

        def fire(t, p):
            col = pl.multiple_of(tile_of(t) * _CHUNK, _CHUNK)
            pltpu.async_copy(
                tblT_hbm.at[:, pl.ds(col, _CHUNK)], srcs[p], sems[p])

        def wait(p):
            pltpu.make_async_copy(
                tblT_hbm.at[:, pl.ds(0, _CHUNK)], srcs[p], sems[p]).wait()

        fire(0, 0)
        fire(1, 1)

        def body(k, _):
            for p in range(2):
                t = 2 * k + p
                wait(p)
                src = srcs[p]

                def tr(rq, _, src=src):
                    r0 = rq * 4
                    for dr in range(4):
                        r = r0 + dr
                        cols = [jnp.full((_LANES,), 0, jnp.int32)
                                + (4 * r + d) for d in range(4)]
                        for m in range(2 * _CHUNK // D):
                            f_vec = f_hi if (m & 1) else f_lo
                            vals = plsc.load_gather(src, [f_vec, cols[m >> 1]])
                            dst_v[r, pl.ds(_LANES * m, _LANES)] = vals
                    return _

                lax.fori_loop(0, D // 4, tr, None)
                row = pl.multiple_of(tile_of(t) * D, 8)
                pltpu.sync_copy(dst_v, cvt_hbm.at[pl.ds(row, D), :])
                fire(t + 2, p)
            return _

        lax.fori_loop(0, (per_w + 1) // 2, body, None)
        wait(0)
        wait(1)

    return convert_kernel


@functools.lru_cache(maxsize=None)
def _make_gather(V, D, B):
    info = plsc.get_sparse_core_info()
    NC, NS = info.num_cores, info.num_subcores
    NW = NC * NS
    b_per_w = B // NW
    n_chunks = b_per_w // _CHUNK
    per_line = _CHUNK // D
    mesh = plsc.VectorSubcoreMesh(core_axis_name="c", subcore_axis_name="s")

    @functools.partial(
        pl.kernel,
        mesh=mesh,
        out_type=jax.ShapeDtypeStruct((D, B), jnp.float32),
        compiler_params=pltpu.CompilerParams(needs_layout_passes=False),
        scratch_types=[
            pltpu.VMEM((b_per_w,), jnp.int32),
            pltpu.VMEM((b_per_w,), jnp.int32),
            pltpu.VMEM((_CHUNK, _CHUNK), jnp.float32),
            pltpu.VMEM((_CHUNK, _CHUNK), jnp.float32),
            pltpu.VMEM((D, b_per_w), jnp.float32),
            pltpu.SemaphoreType.DMA,
            pltpu.SemaphoreType.DMA,
        ],
    )
    def gather_kernel(cvt_hbm, idx_hbm, out_hbm, idx_v, t_v, buf0, buf1,
                      outT_v, sem0, sem1):
        wid = lax.axis_index("s") * NC + lax.axis_index("c")
        base = wid * b_per_w
        pltpu.sync_copy(idx_hbm.at[pl.ds(base, b_per_w)], idx_v)

        def line_ids(g, _):
            off = g * _LANES
            v = idx_v[pl.ds(off, _LANES)]
            t_v[pl.ds(off, _LANES)] = lax.shift_right_logical(v, 2)
            return _

        lax.fori_loop(0, b_per_w // _LANES, line_ids, None)

        bufs = (buf0, buf1)
        sems = (sem0, sem1)

        def fire(j):
            return pltpu.async_copy(
                cvt_hbm.at[t_v.at[pl.ds(j * _CHUNK, _CHUNK)]],
                bufs[j % 2],
                sems[j % 2],
            )

        lane = lax.broadcasted_iota(jnp.int32, (_LANES,), 0)
        copies = {0: fire(0)}
        for j in range(n_chunks):
            if j + 1 < n_chunks:
                copies[j + 1] = fire(j + 1)
            copies.pop(j).wait()
            buf = bufs[j % 2]

            def extract(g, _, j=j, buf=buf):
                off = g * _LANES
                s_vec = lax.bitwise_and(idx_v[pl.ds(off, _LANES)],
                                        per_line - 1)
                pos0 = s_vec * D
                i_vec = lane + g * _LANES - j * _CHUNK
                for c in range(D):
                    vals = plsc.load_gather(buf, [i_vec, pos0 + c])
                    outT_v[c, pl.ds(off, _LANES)] = vals
                return _

            lax.fori_loop(j * (_CHUNK // _LANES), (j + 1) * (_CHUNK // _LANES),
                          extract, None)
        pltpu.sync_copy(outT_v, out_hbm.at[:, pl.ds(base, b_per_w)])

    return gather_kernel


def kernel(labels, table):
    B = labels.shape[0]
    V, D = table.shape
    idx = labels.astype(jnp.int32)
    cvt = _make_convert(V, D)(table.T)
    out_t = _make_gather(V, D, B)(cvt, idx)
    return out_t.T[:, None, :]


# speedup vs baseline: 2.0085x; 2.0085x over previous
import functools

import jax
import jax.numpy as jnp
from jax import lax
from jax.experimental import pallas as pl
from jax.experimental.pallas import tpu as pltpu
from jax.experimental.pallas import tpu_sc as plsc

_CHUNK = 128
_LANES = 16


@functools.lru_cache(maxsize=None)
def _make_convert(V, D):
    info = plsc.get_sparse_core_info()
    NC, NS = info.num_cores, info.num_subcores
    NW = NC * NS
    n_tiles = (V + _CHUNK - 1) // _CHUNK
    per_w = (n_tiles + NW - 1) // NW
    rows_out = n_tiles * D
    mesh = plsc.VectorSubcoreMesh(core_axis_name="c", subcore_axis_name="s")

    @functools.partial(
        pl.kernel,
        mesh=mesh,
        out_type=jax.ShapeDtypeStruct((rows_out, _CHUNK), jnp.float32),
        compiler_params=pltpu.CompilerParams(needs_layout_passes=False),
        scratch_types=[
            pltpu.VMEM((D, _CHUNK), jnp.float32),
            pltpu.VMEM((D, _CHUNK), jnp.float32),
            pltpu.VMEM((D, _CHUNK), jnp.float32),
            pltpu.SemaphoreType.DMA,
            pltpu.SemaphoreType.DMA,
        ],
    )
    def convert_kernel(tblT_hbm, cvt_hbm, src0, src1, dst_v, sem0, sem1):
        wid = lax.axis_index("s") * NC + lax.axis_index("c")
        start = jnp.minimum(wid * per_w, n_tiles - per_w)
        srcs = (src0, src1)
        sems = (sem0, sem1)
        lane = lax.broadcasted_iota(jnp.int32, (_LANES,), 0)
        lane_q = lax.shift_right_logical(lane, 2)
        lane_r3 = lax.shift_left(lax.bitwise_and(lane, 3), 5)

        def tile_of(t):
            return start + jnp.minimum(t, per_w - 1)

        def fire(t, p):
            col = pl.multiple_of(tile_of(t) * _CHUNK, _CHUNK)
            pltpu.async_copy(
                tblT_hbm.at[:, pl.ds(col, _CHUNK)], srcs[p], sems[p])

        def wait(p):
            pltpu.make_async_copy(
                tblT_hbm.at[:, pl.ds(0, _CHUNK)], srcs[p], sems[p]).wait()

        fire(0, 0)
        fire(1, 1)

        def body(k, _):
            for p in range(2):
                t = 2 * k + p
                wait(p)
                src = srcs[p]

                row_base = lane_q
                col_base = lane_r3

                def tr(h, _, src=src):
                    f0 = h * (D // 2)
                    for df in range(D // 2):
                        f = f0 + df

                        def step(q, _, f=f):
                            c0 = q * _LANES
                            vals = src[f, pl.ds(c0, _LANES)]
                            rot = lax.bitwise_and(lane + (f + c0), D - 1)
                            row = row_base + lax.shift_right_logical(c0, 2)
                            col = col_base + rot
                            plsc.store_scatter(dst_v, [row, col], vals)
                            return _

                        lax.fori_loop(0, _CHUNK // _LANES, step, None)
                    return _

                lax.fori_loop(0, 2, tr, None)
                row = pl.multiple_of(tile_of(t) * D, 8)
                pltpu.sync_copy(dst_v, cvt_hbm.at[pl.ds(row, D), :])
                fire(t + 2, p)
            return _

        lax.fori_loop(0, (per_w + 1) // 2, body, None)
        wait(0)
        wait(1)

    return convert_kernel


@functools.lru_cache(maxsize=None)
def _make_gather(V, D, B):
    info = plsc.get_sparse_core_info()
    NC, NS = info.num_cores, info.num_subcores
    NW = NC * NS
    b_per_w = B // NW
    n_chunks = b_per_w // _CHUNK
    per_line = _CHUNK // D
    mesh = plsc.VectorSubcoreMesh(core_axis_name="c", subcore_axis_name="s")

    @functools.partial(
        pl.kernel,
        mesh=mesh,
        out_type=jax.ShapeDtypeStruct((D, B), jnp.float32),
        compiler_params=pltpu.CompilerParams(needs_layout_passes=False),
        scratch_types=[
            pltpu.VMEM((b_per_w,), jnp.int32),
            pltpu.VMEM((b_per_w,), jnp.int32),
            pltpu.VMEM((_CHUNK, _CHUNK), jnp.float32),
            pltpu.VMEM((_CHUNK, _CHUNK), jnp.float32),
            pltpu.VMEM((D, b_per_w), jnp.float32),
            pltpu.SemaphoreType.DMA,
            pltpu.SemaphoreType.DMA,
        ],
    )
    def gather_kernel(cvt_hbm, idx_hbm, out_hbm, idx_v, t_v, buf0, buf1,
                      outT_v, sem0, sem1):
        wid = lax.axis_index("s") * NC + lax.axis_index("c")
        base = wid * b_per_w
        pltpu.sync_copy(idx_hbm.at[pl.ds(base, b_per_w)], idx_v)

        def line_ids(g, _):
            off = g * _LANES
            v = idx_v[pl.ds(off, _LANES)]
            t_v[pl.ds(off, _LANES)] = lax.shift_right_logical(v, 2)
            return _

        lax.fori_loop(0, b_per_w // _LANES, line_ids, None)

        bufs = (buf0, buf1)
        sems = (sem0, sem1)

        def fire(j):
            return pltpu.async_copy(
                cvt_hbm.at[t_v.at[pl.ds(j * _CHUNK, _CHUNK)]],
                bufs[j % 2],
                sems[j % 2],
            )

        lane = lax.broadcasted_iota(jnp.int32, (_LANES,), 0)
        copies = {0: fire(0)}
        for j in range(n_chunks):
            if j + 1 < n_chunks:
                copies[j + 1] = fire(j + 1)
            copies.pop(j).wait()
            buf = bufs[j % 2]

            def extract(g, _, j=j, buf=buf):
                off = g * _LANES
                v_vec = idx_v[pl.ds(off, _LANES)]
                s_vec = lax.bitwise_and(v_vec, per_line - 1)
                pos0 = s_vec * D
                i_vec = lane + g * _LANES - j * _CHUNK
                for c in range(D):
                    rot = lax.bitwise_and(v_vec + c, D - 1)
                    vals = plsc.load_gather(buf, [i_vec, pos0 + rot])
                    outT_v[c, pl.ds(off, _LANES)] = vals
                return _

            lax.fori_loop(j * (_CHUNK // _LANES), (j + 1) * (_CHUNK // _LANES),
                          extract, None)
        pltpu.sync_copy(outT_v, out_hbm.at[:, pl.ds(base, b_per_w)])

    return gather_kernel


def kernel(labels, table):
    B = labels.shape[0]
    V, D = table.shape
    idx = labels.astype(jnp.int32)
    cvt = _make_convert(V, D)(table.T)
    out_t = _make_gather(V, D, B)(cvt, idx)
    return out_t.T[:, None, :]
